# Initial kernel scaffold; baseline (speedup 1.0000x reference)
import functools

import jax
import jax.numpy as jnp
from jax import lax
from jax.experimental import pallas as pl
from jax.experimental.pallas import tpu as pltpu
from jax.experimental.pallas import tpu_sc as plsc

N_NODES = 10000
N_EDGES = 320000
NC, NS = 2, 16
NW = NC * NS
EPT = N_EDGES // NW
CH = 80
NCHUNK = EPT // CH
RPT = N_NODES // NS
ROW1 = 80
ROW2 = 16

_f32 = jnp.float32
_i32 = jnp.int32


def _prep1_body(x_ref, w1_ref, asrc_ref, adst_ref, nodetab_ref, adsttab_ref):
    h = jnp.dot(x_ref[...], w1_ref[...], preferred_element_type=_f32)
    row = lax.broadcasted_iota(_i32, (64, 8), 0)
    col = lax.broadcasted_iota(_i32, (64, 8), 1)
    g = (row // 8 == col).astype(_f32)
    asrc = jnp.dot(h * asrc_ref[...], g, preferred_element_type=_f32)
    adst = jnp.dot(h * adst_ref[...], g, preferred_element_type=_f32)
    z8 = jnp.zeros_like(asrc)
    nodetab_ref[...] = jnp.concatenate([asrc, z8, h], axis=1)
    adsttab_ref[...] = jnp.concatenate([adst, z8], axis=1)


_prep1 = pl.pallas_call(
    _prep1_body,
    out_shape=(
        jax.ShapeDtypeStruct((N_NODES, ROW1), _f32),
        jax.ShapeDtypeStruct((N_NODES, 16), _f32),
    ),
)


def _sc_edge1_body(nodetab, adsttab, srcidx, dstidx, out_hbm,
                   sidx, didx, srows, orows, drows, wtmp, stage, accum,
                   sem0, sem1):
    cid = lax.axis_index("c")
    tid = lax.axis_index("s")
    base = (cid * NS + tid) * EPT
    iota = lax.broadcasted_iota(_i32, (16,), 0)
    zeros16 = jnp.zeros((16,), _f32)

    def zrow(r, c):
        for k in range(ROW1 // 16):
            stage[r, pl.ds(k * 16, 16)] = zeros16
        return c
    lax.fori_loop(0, RPT, zrow, 0)
    pltpu.sync_copy(stage, accum.at[pl.ds(tid * RPT, RPT)])
    plsc.subcore_barrier()

    def chunk(ci, c):
        off = base + ci * CH
        pltpu.sync_copy(srcidx.at[pl.ds(off, CH)], sidx)
        pltpu.sync_copy(dstidx.at[pl.ds(off, CH)], didx)
        cp0 = pltpu.async_copy(nodetab.at[sidx], srows, sem0)
        cp1 = pltpu.async_copy(adsttab.at[didx], drows, sem1)
        cp0.wait()
        cp1.wait()

        def edge(i, c2):
            r0 = srows[i, pl.ds(0, 16)]
            d0 = drows[i, pl.ds(0, 16)]
            t = r0 + d0
            t = jnp.where(t > 0, t, 0.2 * t)
            w = jnp.exp(t)
            orows[i, pl.ds(0, 16)] = w
            wtmp[...] = w
            for k in range(1, ROW1 // 16):
                colk = 2 * (k - 1) + (iota >= 8).astype(_i32)
                wk = plsc.load_gather(wtmp, [colk])
                orows[i, pl.ds(16 * k, 16)] = wk * srows[i, pl.ds(16 * k, 16)]
            return c2
        lax.fori_loop(0, CH, edge, 0)
        pltpu.sync_copy(orows, accum.at[didx], add=True)
        return c
    lax.fori_loop(0, NCHUNK, chunk, 0)

    plsc.subcore_barrier()
    pltpu.sync_copy(accum.at[pl.ds(tid * RPT, RPT)], stage)
    pltpu.sync_copy(stage, out_hbm.at[cid, pl.ds(tid * RPT, RPT)])


def _make_sc_edge1(interpret=False):
    return functools.partial(
        pl.kernel,
        out_type=jax.ShapeDtypeStruct((NC, N_NODES, ROW1), _f32),
        mesh=plsc.VectorSubcoreMesh(core_axis_name="c", subcore_axis_name="s",
                                    num_cores=NC, num_subcores=NS),
        scratch_types=[
            pltpu.VMEM((CH,), _i32),
            pltpu.VMEM((CH,), _i32),
            pltpu.VMEM((CH, ROW1), _f32),
            pltpu.VMEM((CH, ROW1), _f32),
            pltpu.VMEM((CH, 16), _f32),
            pltpu.VMEM((16,), _f32),
            pltpu.VMEM((RPT, ROW1), _f32),
            pltpu.VMEM_SHARED((N_NODES, ROW1), _f32),
            pltpu.SemaphoreType.DMA,
            pltpu.SemaphoreType.DMA,
        ],
        interpret=interpret,
    )(_sc_edge1_body)


_sc_edge1 = _make_sc_edge1()


def _mid_body(part_ref, nodetab_ref, adsttab_ref, b1_ref, w2_ref,
              as2_ref, ad2_ref, nodetab2_ref):
    acc = part_ref[0] + part_ref[1]
    nodetab = nodetab_ref[...]
    asrc = nodetab[:, 0:8]
    h1 = nodetab[:, 16:ROW1]
    adst = adsttab_ref[...][:, 0:8]
    t = asrc + adst
    wself = jnp.exp(jnp.where(t > 0, t, 0.2 * t))
    den = acc[:, 0:8] + wself
    row = lax.broadcasted_iota(_i32, (8, 64), 0)
    col = lax.broadcasted_iota(_i32, (8, 64), 1)
    s8 = (row == col // 8).astype(_f32)
    num = acc[:, 16:ROW1] + jnp.dot(wself, s8, preferred_element_type=_f32) * h1
    dene = jnp.dot(den, s8, preferred_element_type=_f32)
    o = num / (dene + 1e-16) + b1_ref[...]
    x2 = jnp.where(o > 0, o, jnp.exp(o) - 1.0)
    h2 = jnp.dot(x2, w2_ref[...], preferred_element_type=_f32)
    as2 = jnp.dot(h2, as2_ref[...], preferred_element_type=_f32)
    ad2 = jnp.dot(h2, ad2_ref[...], preferred_element_type=_f32)
    z11 = jnp.zeros((h2.shape[0], 11), _f32)
    nodetab2_ref[...] = jnp.concatenate([as2, h2, ad2, z11], axis=1)


_mid = pl.pallas_call(
    _mid_body,
    out_shape=jax.ShapeDtypeStruct((N_NODES, ROW2), _f32),
)


def _sc_edge2_body(nodetab2, srcidx, dstidx, out_hbm,
                   sidx, didx, srows, drows, orows, wtmp, stage, accum,
                   sem0, sem1):
    cid = lax.axis_index("c")
    tid = lax.axis_index("s")
    base = (cid * NS + tid) * EPT
    iota = lax.broadcasted_iota(_i32, (16,), 0)
    zeros16 = jnp.zeros((16,), _f32)
    zidx = jnp.zeros((16,), _i32)

    def zrow(r, c):
        stage[r, pl.ds(0, 16)] = zeros16
        return c
    lax.fori_loop(0, RPT, zrow, 0)
    pltpu.sync_copy(stage, accum.at[pl.ds(tid * RPT, RPT)])
    plsc.subcore_barrier()

    def chunk(ci, c):
        off = base + ci * CH
        pltpu.sync_copy(srcidx.at[pl.ds(off, CH)], sidx)
        pltpu.sync_copy(dstidx.at[pl.ds(off, CH)], didx)
        cp0 = pltpu.async_copy(nodetab2.at[sidx], srows, sem0)
        cp1 = pltpu.async_copy(nodetab2.at[didx], drows, sem1)
        cp0.wait()
        cp1.wait()

        def edge(i, c2):
            s = srows[i, pl.ds(0, 16)]
            irow = jnp.full((16,), i, _i32)
            advec = plsc.load_gather(drows, [irow, jnp.full((16,), 4, _i32)])
            t = s + advec
            t = jnp.where(t > 0, t, 0.2 * t)
            w = jnp.exp(t)
            wtmp[...] = w
            wb = plsc.load_gather(wtmp, [zidx])
            sm = jnp.where(iota == 0, 1.0, jnp.where(iota < 4, s, 0.0))
            orows[i, pl.ds(0, 16)] = wb * sm
            return c2
        lax.fori_loop(0, CH, edge, 0)
        pltpu.sync_copy(orows, accum.at[didx], add=True)
        return c
    lax.fori_loop(0, NCHUNK, chunk, 0)

    plsc.subcore_barrier()
    pltpu.sync_copy(accum.at[pl.ds(tid * RPT, RPT)], stage)
    pltpu.sync_copy(stage, out_hbm.at[cid, pl.ds(tid * RPT, RPT)])


def _make_sc_edge2(interpret=False):
    return functools.partial(
        pl.kernel,
        out_type=jax.ShapeDtypeStruct((NC, N_NODES, ROW2), _f32),
        mesh=plsc.VectorSubcoreMesh(core_axis_name="c", subcore_axis_name="s",
                                    num_cores=NC, num_subcores=NS),
        scratch_types=[
            pltpu.VMEM((CH,), _i32),
            pltpu.VMEM((CH,), _i32),
            pltpu.VMEM((CH, ROW2), _f32),
            pltpu.VMEM((CH, ROW2), _f32),
            pltpu.VMEM((CH, ROW2), _f32),
            pltpu.VMEM((16,), _f32),
            pltpu.VMEM((RPT, ROW2), _f32),
            pltpu.VMEM_SHARED((N_NODES, ROW2), _f32),
            pltpu.SemaphoreType.DMA,
            pltpu.SemaphoreType.DMA,
        ],
        interpret=interpret,
    )(_sc_edge2_body)


_sc_edge2 = _make_sc_edge2()


def _fin_body(part2_ref, nodetab2_ref, b2_ref, out_ref):
    acc = part2_ref[0] + part2_ref[1]
    tab = nodetab2_ref[...]
    as2 = tab[:, 0:1]
    h2 = tab[:, 1:4]
    ad2 = tab[:, 4:5]
    t = as2 + ad2
    w = jnp.exp(jnp.where(t > 0, t, 0.2 * t))
    den = acc[:, 0:1] + w
    num = acc[:, 1:4] + w * h2
    out_ref[...] = num / (den + 1e-16) + b2_ref[...]


_fin = pl.pallas_call(
    _fin_body,
    out_shape=jax.ShapeDtypeStruct((N_NODES, 3), _f32),
)


@jax.jit
def kernel(x, edge_index, W1, att_src1, att_dst1, b1, W2, att_src2, att_dst2, b2):
    ei = edge_index.astype(_i32)
    src, dst = ei[0], ei[1]
    nodetab, adsttab = _prep1(x, W1, att_src1.reshape(1, 64),
                              att_dst1.reshape(1, 64))
    part1 = _sc_edge1(nodetab, adsttab, src, dst)
    nodetab2 = _mid(part1, nodetab, adsttab, b1.reshape(1, 64), W2,
                    att_src2.reshape(3, 1), att_dst2.reshape(3, 1))
    part2 = _sc_edge2(nodetab2, src, dst)
    return _fin(part2, nodetab2, b2.reshape(1, 3))


# speedup vs baseline: 57.9911x; 57.9911x over previous
import functools

import jax
import jax.numpy as jnp
from jax import lax
from jax.experimental import pallas as pl
from jax.experimental.pallas import tpu as pltpu
from jax.experimental.pallas import tpu_sc as plsc

N_NODES = 10000
NPAD = 10240
N_EDGES = 320000
NC, NS = 2, 16
NW = NC * NS
EPT = N_EDGES // NW
CH = 80
NCHUNK = EPT // CH
RPT = NPAD // NS
ROW1 = 80
ROW2 = 16

_f32 = jnp.float32
_i32 = jnp.int32


def _prep1_body(x_ref, w1_ref, asrc_ref, adst_ref, nodetab_ref, adsttab_ref):
    h = jnp.dot(x_ref[...], w1_ref[...], preferred_element_type=_f32)
    row = lax.broadcasted_iota(_i32, (64, 8), 0)
    col = lax.broadcasted_iota(_i32, (64, 8), 1)
    g = (row // 8 == col).astype(_f32)
    asrc = jnp.dot(h * asrc_ref[...], g, preferred_element_type=_f32)
    adst = jnp.dot(h * adst_ref[...], g, preferred_element_type=_f32)
    z8 = jnp.zeros_like(asrc)
    nodetab_ref[...] = jnp.concatenate([asrc, z8, h], axis=1)
    adsttab_ref[...] = jnp.concatenate([adst, z8], axis=1)


_prep1 = pl.pallas_call(
    _prep1_body,
    out_shape=(
        jax.ShapeDtypeStruct((N_NODES, ROW1), _f32),
        jax.ShapeDtypeStruct((N_NODES, 16), _f32),
    ),
)


def _sc_edge1_body(nodetab, adsttab, srcidx, dstidx, out_hbm,
                   sidx, didx, srows, orows, drows, stage, accum,
                   sem0, sem1):
    cid = lax.axis_index("c")
    tid = lax.axis_index("s")
    base = (cid * NS + tid) * EPT
    iota = lax.broadcasted_iota(_i32, (16,), 0)
    zeros16 = jnp.zeros((16,), _f32)

    def zrow(r, c):
        for k in range(ROW1 // 16):
            stage[r, pl.ds(k * 16, 16)] = zeros16
        return c
    lax.fori_loop(0, RPT, zrow, 0)
    pltpu.sync_copy(stage, accum.at[pl.ds(tid * RPT, RPT)])
    plsc.subcore_barrier()

    def chunk(ci, c):
        off = base + ci * CH
        pltpu.sync_copy(srcidx.at[pl.ds(off, CH)], sidx)
        pltpu.sync_copy(dstidx.at[pl.ds(off, CH)], didx)
        cp0 = pltpu.async_copy(nodetab.at[sidx], srows, sem0)
        cp1 = pltpu.async_copy(adsttab.at[didx], drows, sem1)
        cp0.wait()
        cp1.wait()

        def edge(i, c2):
            r0 = srows[i, pl.ds(0, 16)]
            d0 = drows[i, pl.ds(0, 16)]
            t = r0 + d0
            t = jnp.where(t > 0, t, 0.2 * t)
            w = jnp.exp(t)
            orows[i, pl.ds(0, 16)] = w
            for k in range(1, ROW1 // 16):
                colk = iota // 8 + 2 * (k - 1)
                wk = w.at[colk].get(mode="promise_in_bounds")
                orows[i, pl.ds(16 * k, 16)] = wk * srows[i, pl.ds(16 * k, 16)]
            return c2
        lax.fori_loop(0, CH, edge, 0)
        pltpu.sync_copy(orows, accum.at[didx], add=True)
        return c
    lax.fori_loop(0, NCHUNK, chunk, 0)

    plsc.subcore_barrier()
    pltpu.sync_copy(accum.at[pl.ds(tid * RPT, RPT)], stage)
    pltpu.sync_copy(stage, out_hbm.at[cid, pl.ds(tid * RPT, RPT)])


def _make_sc_edge1(interpret=False):
    return functools.partial(
        pl.kernel,
        out_type=jax.ShapeDtypeStruct((NC, NPAD, ROW1), _f32),
        mesh=plsc.VectorSubcoreMesh(core_axis_name="c", subcore_axis_name="s",
                                    num_cores=NC, num_subcores=NS),
        scratch_types=[
            pltpu.VMEM((CH,), _i32),
            pltpu.VMEM((CH,), _i32),
            pltpu.VMEM((CH, ROW1), _f32),
            pltpu.VMEM((CH, ROW1), _f32),
            pltpu.VMEM((CH, 16), _f32),
            pltpu.VMEM((RPT, ROW1), _f32),
            pltpu.VMEM_SHARED((NPAD, ROW1), _f32),
            pltpu.SemaphoreType.DMA,
            pltpu.SemaphoreType.DMA,
        ],
        compiler_params=pltpu.CompilerParams(use_tc_tiling_on_sc=False,
                                              needs_layout_passes=False),
        interpret=interpret,
    )(_sc_edge1_body)


_sc_edge1 = _make_sc_edge1()


def _mid_body(part_ref, nodetab_ref, adsttab_ref, b1_ref, w2_ref,
              as2_ref, ad2_ref, nodetab2_ref):
    acc = part_ref[0, :N_NODES] + part_ref[1, :N_NODES]
    nodetab = nodetab_ref[...]
    asrc = nodetab[:, 0:8]
    h1 = nodetab[:, 16:ROW1]
    adst = adsttab_ref[...][:, 0:8]
    t = asrc + adst
    wself = jnp.exp(jnp.where(t > 0, t, 0.2 * t))
    den = acc[:, 0:8] + wself
    row = lax.broadcasted_iota(_i32, (8, 64), 0)
    col = lax.broadcasted_iota(_i32, (8, 64), 1)
    s8 = (row == col // 8).astype(_f32)
    num = acc[:, 16:ROW1] + jnp.dot(wself, s8, preferred_element_type=_f32) * h1
    dene = jnp.dot(den, s8, preferred_element_type=_f32)
    o = num / (dene + 1e-16) + b1_ref[...]
    x2 = jnp.where(o > 0, o, jnp.exp(o) - 1.0)
    h2 = jnp.dot(x2, w2_ref[...], preferred_element_type=_f32)
    as2 = jnp.dot(h2, as2_ref[...], preferred_element_type=_f32)
    ad2 = jnp.dot(h2, ad2_ref[...], preferred_element_type=_f32)
    z11 = jnp.zeros((h2.shape[0], 11), _f32)
    nodetab2_ref[...] = jnp.concatenate([as2, h2, ad2, z11], axis=1)


_mid = pl.pallas_call(
    _mid_body,
    out_shape=jax.ShapeDtypeStruct((N_NODES, ROW2), _f32),
)


def _sc_edge2_body(nodetab2, srcidx, dstidx, out_hbm,
                   sidx, didx, srows, drows, orows, stage, accum,
                   sem0, sem1):
    cid = lax.axis_index("c")
    tid = lax.axis_index("s")
    base = (cid * NS + tid) * EPT
    iota = lax.broadcasted_iota(_i32, (16,), 0)
    zeros16 = jnp.zeros((16,), _f32)
    zidx = jnp.zeros((16,), _i32)

    def zrow(r, c):
        stage[r, pl.ds(0, 16)] = zeros16
        return c
    lax.fori_loop(0, RPT, zrow, 0)
    pltpu.sync_copy(stage, accum.at[pl.ds(tid * RPT, RPT)])
    plsc.subcore_barrier()

    def chunk(ci, c):
        off = base + ci * CH
        pltpu.sync_copy(srcidx.at[pl.ds(off, CH)], sidx)
        pltpu.sync_copy(dstidx.at[pl.ds(off, CH)], didx)
        cp0 = pltpu.async_copy(nodetab2.at[sidx], srows, sem0)
        cp1 = pltpu.async_copy(nodetab2.at[didx], drows, sem1)
        cp0.wait()
        cp1.wait()

        def edge(i, c2):
            s = srows[i, pl.ds(0, 16)]
            irow = jnp.full((16,), i, _i32)
            advec = plsc.load_gather(drows, [irow, jnp.full((16,), 4, _i32)])
            t = s + advec
            t = jnp.where(t > 0, t, 0.2 * t)
            w = jnp.exp(t)
            wb = w.at[zidx].get(mode="promise_in_bounds")
            sm = jnp.where(iota == 0, 1.0, jnp.where(iota < 4, s, 0.0))
            orows[i, pl.ds(0, 16)] = wb * sm
            return c2
        lax.fori_loop(0, CH, edge, 0)
        pltpu.sync_copy(orows, accum.at[didx], add=True)
        return c
    lax.fori_loop(0, NCHUNK, chunk, 0)

    plsc.subcore_barrier()
    pltpu.sync_copy(accum.at[pl.ds(tid * RPT, RPT)], stage)
    pltpu.sync_copy(stage, out_hbm.at[cid, pl.ds(tid * RPT, RPT)])


def _make_sc_edge2(interpret=False):
    return functools.partial(
        pl.kernel,
        out_type=jax.ShapeDtypeStruct((NC, NPAD, ROW2), _f32),
        mesh=plsc.VectorSubcoreMesh(core_axis_name="c", subcore_axis_name="s",
                                    num_cores=NC, num_subcores=NS),
        scratch_types=[
            pltpu.VMEM((CH,), _i32),
            pltpu.VMEM((CH,), _i32),
            pltpu.VMEM((CH, ROW2), _f32),
            pltpu.VMEM((CH, ROW2), _f32),
            pltpu.VMEM((CH, ROW2), _f32),
            pltpu.VMEM((RPT, ROW2), _f32),
            pltpu.VMEM_SHARED((NPAD, ROW2), _f32),
            pltpu.SemaphoreType.DMA,
            pltpu.SemaphoreType.DMA,
        ],
        compiler_params=pltpu.CompilerParams(use_tc_tiling_on_sc=False,
                                              needs_layout_passes=False),
        interpret=interpret,
    )(_sc_edge2_body)


_sc_edge2 = _make_sc_edge2()


def _fin_body(part2_ref, nodetab2_ref, b2_ref, out_ref):
    acc = part2_ref[0, :N_NODES] + part2_ref[1, :N_NODES]
    tab = nodetab2_ref[...]
    as2 = tab[:, 0:1]
    h2 = tab[:, 1:4]
    ad2 = tab[:, 4:5]
    t = as2 + ad2
    w = jnp.exp(jnp.where(t > 0, t, 0.2 * t))
    den = acc[:, 0:1] + w
    num = acc[:, 1:4] + w * h2
    out_ref[...] = num / (den + 1e-16) + b2_ref[...]


_fin = pl.pallas_call(
    _fin_body,
    out_shape=jax.ShapeDtypeStruct((N_NODES, 3), _f32),
)


@jax.jit
def kernel(x, edge_index, W1, att_src1, att_dst1, b1, W2, att_src2, att_dst2, b2):
    ei = edge_index.astype(_i32)
    src, dst = ei[0], ei[1]
    nodetab, adsttab = _prep1(x, W1, att_src1.reshape(1, 64),
                              att_dst1.reshape(1, 64))
    part1 = _sc_edge1(nodetab, adsttab, src, dst)
    nodetab2 = _mid(part1, nodetab, adsttab, b1.reshape(1, 64), W2,
                    att_src2.reshape(3, 1), att_dst2.reshape(3, 1))
    part2 = _sc_edge2(nodetab2, src, dst)
    return _fin(part2, nodetab2, b2.reshape(1, 3))


# speedup vs baseline: 82.9202x; 1.4299x over previous
import functools

import jax
import jax.numpy as jnp
from jax import lax
from jax.experimental import pallas as pl
from jax.experimental.pallas import tpu as pltpu
from jax.experimental.pallas import tpu_sc as plsc

N_NODES = 10000
NPAD = 10240
N_EDGES = 320000
NC, NS = 2, 16
NW = NC * NS
EPT = N_EDGES // NW
CH = 80
NCHUNK = EPT // CH
RPT = NPAD // NS
ROW1 = 80
ROW2 = 16

_f32 = jnp.float32
_i32 = jnp.int32


def _prep1_body(x_ref, w1_ref, asrc_ref, adst_ref, nodetab_ref, adsttab_ref):
    h = jnp.dot(x_ref[...], w1_ref[...], preferred_element_type=_f32)
    row = lax.broadcasted_iota(_i32, (64, 8), 0)
    col = lax.broadcasted_iota(_i32, (64, 8), 1)
    g = (row // 8 == col).astype(_f32)
    asrc = jnp.dot(h * asrc_ref[...], g, preferred_element_type=_f32)
    adst = jnp.dot(h * adst_ref[...], g, preferred_element_type=_f32)
    z8 = jnp.zeros_like(asrc)
    nodetab_ref[...] = jnp.concatenate([asrc, z8, h], axis=1)
    adsttab_ref[...] = jnp.concatenate([adst, z8], axis=1)


_prep1 = pl.pallas_call(
    _prep1_body,
    out_shape=(
        jax.ShapeDtypeStruct((N_NODES, ROW1), _f32),
        jax.ShapeDtypeStruct((N_NODES, 16), _f32),
    ),
)


def _sc_edge1_body(nodetab, adsttab, srcidx, dstidx, out_hbm,
                   sidx, didx, srows, orows, drows, stage, accum,
                   sem0, sem1):
    cid = lax.axis_index("c")
    tid = lax.axis_index("s")
    base = (cid * NS + tid) * EPT
    iota = lax.broadcasted_iota(_i32, (16,), 0)
    zeros16 = jnp.zeros((16,), _f32)

    def zrow(r, c):
        for k in range(ROW1 // 16):
            stage[r, pl.ds(k * 16, 16)] = zeros16
        return c
    lax.fori_loop(0, RPT, zrow, 0)
    pltpu.sync_copy(stage, accum.at[pl.ds(tid * RPT, RPT)])
    plsc.subcore_barrier()

    def chunk(ci, c):
        off = base + ci * CH
        pltpu.sync_copy(srcidx.at[pl.ds(off, CH)], sidx)
        pltpu.sync_copy(dstidx.at[pl.ds(off, CH)], didx)
        cp0 = pltpu.async_copy(nodetab.at[sidx], srows, sem0)
        cp1 = pltpu.async_copy(adsttab.at[didx], drows, sem1)
        cp0.wait()
        cp1.wait()

        @plsc.parallel_loop(0, CH, unroll=4)
        def edge(i):
            r0 = srows[i, pl.ds(0, 16)]
            d0 = drows[i, pl.ds(0, 16)]
            t = r0 + d0
            t = jnp.where(t > 0, t, 0.2 * t)
            w = jnp.exp(t)
            orows[i, pl.ds(0, 16)] = w
            for k in range(1, ROW1 // 16):
                colk = iota // 8 + 2 * (k - 1)
                wk = w.at[colk].get(mode="promise_in_bounds")
                orows[i, pl.ds(16 * k, 16)] = wk * srows[i, pl.ds(16 * k, 16)]
        pltpu.sync_copy(orows, accum.at[didx], add=True)
        return c
    lax.fori_loop(0, NCHUNK, chunk, 0)

    plsc.subcore_barrier()
    pltpu.sync_copy(accum.at[pl.ds(tid * RPT, RPT)], stage)
    pltpu.sync_copy(stage, out_hbm.at[cid, pl.ds(tid * RPT, RPT)])


def _make_sc_edge1(interpret=False):
    return functools.partial(
        pl.kernel,
        out_type=jax.ShapeDtypeStruct((NC, NPAD, ROW1), _f32),
        mesh=plsc.VectorSubcoreMesh(core_axis_name="c", subcore_axis_name="s",
                                    num_cores=NC, num_subcores=NS),
        scratch_types=[
            pltpu.VMEM((CH,), _i32),
            pltpu.VMEM((CH,), _i32),
            pltpu.VMEM((CH, ROW1), _f32),
            pltpu.VMEM((CH, ROW1), _f32),
            pltpu.VMEM((CH, 16), _f32),
            pltpu.VMEM((RPT, ROW1), _f32),
            pltpu.VMEM_SHARED((NPAD, ROW1), _f32),
            pltpu.SemaphoreType.DMA,
            pltpu.SemaphoreType.DMA,
        ],
        compiler_params=pltpu.CompilerParams(use_tc_tiling_on_sc=False,
                                              needs_layout_passes=False),
        interpret=interpret,
    )(_sc_edge1_body)


_sc_edge1 = _make_sc_edge1()


def _mid_body(part_ref, nodetab_ref, adsttab_ref, b1_ref, w2_ref,
              as2_ref, ad2_ref, nodetab2_ref):
    acc = part_ref[0, :N_NODES] + part_ref[1, :N_NODES]
    nodetab = nodetab_ref[...]
    asrc = nodetab[:, 0:8]
    h1 = nodetab[:, 16:ROW1]
    adst = adsttab_ref[...][:, 0:8]
    t = asrc + adst
    wself = jnp.exp(jnp.where(t > 0, t, 0.2 * t))
    den = acc[:, 0:8] + wself
    row = lax.broadcasted_iota(_i32, (8, 64), 0)
    col = lax.broadcasted_iota(_i32, (8, 64), 1)
    s8 = (row == col // 8).astype(_f32)
    num = acc[:, 16:ROW1] + jnp.dot(wself, s8, preferred_element_type=_f32) * h1
    dene = jnp.dot(den, s8, preferred_element_type=_f32)
    o = num / (dene + 1e-16) + b1_ref[...]
    x2 = jnp.where(o > 0, o, jnp.exp(o) - 1.0)
    h2 = jnp.dot(x2, w2_ref[...], preferred_element_type=_f32)
    as2 = jnp.dot(h2, as2_ref[...], preferred_element_type=_f32)
    ad2 = jnp.dot(h2, ad2_ref[...], preferred_element_type=_f32)
    z11 = jnp.zeros((h2.shape[0], 11), _f32)
    nodetab2_ref[...] = jnp.concatenate([as2, h2, ad2, z11], axis=1)


_mid = pl.pallas_call(
    _mid_body,
    out_shape=jax.ShapeDtypeStruct((N_NODES, ROW2), _f32),
)


def _sc_edge2_body(nodetab2, srcidx, dstidx, out_hbm,
                   sidx, didx, srows, drows, orows, stage, accum,
                   sem0, sem1):
    cid = lax.axis_index("c")
    tid = lax.axis_index("s")
    base = (cid * NS + tid) * EPT
    iota = lax.broadcasted_iota(_i32, (16,), 0)
    zeros16 = jnp.zeros((16,), _f32)
    zidx = jnp.zeros((16,), _i32)

    def zrow(r, c):
        stage[r, pl.ds(0, 16)] = zeros16
        return c
    lax.fori_loop(0, RPT, zrow, 0)
    pltpu.sync_copy(stage, accum.at[pl.ds(tid * RPT, RPT)])
    plsc.subcore_barrier()

    def chunk(ci, c):
        off = base + ci * CH
        pltpu.sync_copy(srcidx.at[pl.ds(off, CH)], sidx)
        pltpu.sync_copy(dstidx.at[pl.ds(off, CH)], didx)
        cp0 = pltpu.async_copy(nodetab2.at[sidx], srows, sem0)
        cp1 = pltpu.async_copy(nodetab2.at[didx], drows, sem1)
        cp0.wait()
        cp1.wait()

        @plsc.parallel_loop(0, CH, unroll=8)
        def edge(i):
            s = srows[i, pl.ds(0, 16)]
            d = drows[i, pl.ds(0, 16)]
            advec = d.at[jnp.full((16,), 4, _i32)].get(mode="promise_in_bounds")
            t = s + advec
            t = jnp.where(t > 0, t, 0.2 * t)
            w = jnp.exp(t)
            wb = w.at[zidx].get(mode="promise_in_bounds")
            sm = jnp.where(iota == 0, 1.0, jnp.where(iota < 4, s, 0.0))
            orows[i, pl.ds(0, 16)] = wb * sm
        pltpu.sync_copy(orows, accum.at[didx], add=True)
        return c
    lax.fori_loop(0, NCHUNK, chunk, 0)

    plsc.subcore_barrier()
    pltpu.sync_copy(accum.at[pl.ds(tid * RPT, RPT)], stage)
    pltpu.sync_copy(stage, out_hbm.at[cid, pl.ds(tid * RPT, RPT)])


def _make_sc_edge2(interpret=False):
    return functools.partial(
        pl.kernel,
        out_type=jax.ShapeDtypeStruct((NC, NPAD, ROW2), _f32),
        mesh=plsc.VectorSubcoreMesh(core_axis_name="c", subcore_axis_name="s",
                                    num_cores=NC, num_subcores=NS),
        scratch_types=[
            pltpu.VMEM((CH,), _i32),
            pltpu.VMEM((CH,), _i32),
            pltpu.VMEM((CH, ROW2), _f32),
            pltpu.VMEM((CH, ROW2), _f32),
            pltpu.VMEM((CH, ROW2), _f32),
            pltpu.VMEM((RPT, ROW2), _f32),
            pltpu.VMEM_SHARED((NPAD, ROW2), _f32),
            pltpu.SemaphoreType.DMA,
            pltpu.SemaphoreType.DMA,
        ],
        compiler_params=pltpu.CompilerParams(use_tc_tiling_on_sc=False,
                                              needs_layout_passes=False),
        interpret=interpret,
    )(_sc_edge2_body)


_sc_edge2 = _make_sc_edge2()


def _fin_body(part2_ref, nodetab2_ref, b2_ref, out_ref):
    acc = part2_ref[0, :N_NODES] + part2_ref[1, :N_NODES]
    tab = nodetab2_ref[...]
    as2 = tab[:, 0:1]
    h2 = tab[:, 1:4]
    ad2 = tab[:, 4:5]
    t = as2 + ad2
    w = jnp.exp(jnp.where(t > 0, t, 0.2 * t))
    den = acc[:, 0:1] + w
    num = acc[:, 1:4] + w * h2
    out_ref[...] = num / (den + 1e-16) + b2_ref[...]


_fin = pl.pallas_call(
    _fin_body,
    out_shape=jax.ShapeDtypeStruct((N_NODES, 3), _f32),
)


@jax.jit
def kernel(x, edge_index, W1, att_src1, att_dst1, b1, W2, att_src2, att_dst2, b2):
    ei = edge_index.astype(_i32)
    src, dst = ei[0], ei[1]
    nodetab, adsttab = _prep1(x, W1, att_src1.reshape(1, 64),
                              att_dst1.reshape(1, 64))
    part1 = _sc_edge1(nodetab, adsttab, src, dst)
    nodetab2 = _mid(part1, nodetab, adsttab, b1.reshape(1, 64), W2,
                    att_src2.reshape(3, 1), att_dst2.reshape(3, 1))
    part2 = _sc_edge2(nodetab2, src, dst)
    return _fin(part2, nodetab2, b2.reshape(1, 3))


# speedup vs baseline: 196.4941x; 2.3697x over previous
import functools

import jax
import jax.numpy as jnp
from jax import lax
from jax.experimental import pallas as pl
from jax.experimental.pallas import tpu as pltpu
from jax.experimental.pallas import tpu_sc as plsc

N_NODES = 10000
NPAD = 10240
N_EDGES = 320000
NC, NS = 2, 16
NW = NC * NS
EPT = N_EDGES // NW
CH = 80
NCHUNK = EPT // CH
NROWS = N_EDGES // CH
RPT = NPAD // NS
ROW1 = 80
ROW2 = 16

_f32 = jnp.float32
_i32 = jnp.int32

_SC_PARAMS = pltpu.CompilerParams(use_tc_tiling_on_sc=False,
                                  needs_layout_passes=False)


def _prep1_body(x_ref, w1_ref, asrc_ref, adst_ref, nodetab_ref, adsttab_ref):
    h = jnp.dot(x_ref[...], w1_ref[...], preferred_element_type=_f32)
    row = lax.broadcasted_iota(_i32, (64, 8), 0)
    col = lax.broadcasted_iota(_i32, (64, 8), 1)
    g = (row // 8 == col).astype(_f32)
    asrc = jnp.dot(h * asrc_ref[...], g, preferred_element_type=_f32)
    adst = jnp.dot(h * adst_ref[...], g, preferred_element_type=_f32)
    z8 = jnp.zeros_like(asrc)
    nodetab_ref[...] = jnp.concatenate([asrc, z8, h], axis=1)
    adsttab_ref[...] = jnp.concatenate([adst, z8], axis=1)


_prep1 = pl.pallas_call(
    _prep1_body,
    out_shape=(
        jax.ShapeDtypeStruct((N_NODES, ROW1), _f32),
        jax.ShapeDtypeStruct((N_NODES, 16), _f32),
    ),
)


def _zero_stripe(zbuf, accum, tid, rowbytes16):
    zeros16 = jnp.zeros((16,), _f32)

    def zrow(r, c):
        for k in range(rowbytes16):
            zbuf[r, pl.ds(k * 16, 16)] = zeros16
        return c
    lax.fori_loop(0, CH, zrow, 0)
    for j in range(RPT // CH):
        pltpu.sync_copy(zbuf, accum.at[pl.ds(tid * RPT + j * CH, CH)])


def _sc_edge1_body(nodetab, adsttab, srcidx, dstidx, out_hbm,
                   sidx, didx, srows0, srows1, orows0, orows1, drows0, drows1,
                   zbuf, accum,
                   sga0, sga1, sgb0, sgb1, ssc0, ssc1):
    cid = lax.axis_index("c")
    tid = lax.axis_index("s")
    crow = (cid * NS + tid) * NCHUNK
    iota = lax.broadcasted_iota(_i32, (16,), 0)
    srows = (srows0, srows1)
    orows = (orows0, orows1)
    drows = (drows0, drows1)
    sga = (sga0, sga1)
    sgb = (sgb0, sgb1)
    ssc = (ssc0, ssc1)

    _zero_stripe(zbuf, accum, tid, ROW1 // 16)
    plsc.subcore_barrier()

    pltpu.sync_copy(srcidx.at[pl.ds(crow, NCHUNK)], sidx)
    pltpu.sync_copy(dstidx.at[pl.ds(crow, NCHUNK)], didx)

    def start_gather(ci, b):
        pltpu.async_copy(nodetab.at[sidx.at[ci]], srows[b], sga[b])
        pltpu.async_copy(adsttab.at[didx.at[ci]], drows[b], sgb[b])

    def wait_gather(b):
        pltpu.make_async_copy(nodetab.at[sidx.at[0]], srows[b], sga[b]).wait()
        pltpu.make_async_copy(adsttab.at[didx.at[0]], drows[b], sgb[b]).wait()

    def start_scatter(ci, b):
        pltpu.async_copy(orows[b], accum.at[didx.at[ci]], ssc[b], add=True)

    def wait_scatter(b):
        pltpu.make_async_copy(orows[b], accum.at[didx.at[0]], ssc[b]).wait()

    def compute(ci, b):
        sr, orow, dr = srows[b], orows[b], drows[b]

        @plsc.parallel_loop(0, CH, unroll=4)
        def edge(i):
            r0 = sr[i, pl.ds(0, 16)]
            d0 = dr[i, pl.ds(0, 16)]
            t = r0 + d0
            t = jnp.where(t > 0, t, 0.2 * t)
            w = jnp.exp(t)
            orow[i, pl.ds(0, 16)] = w
            for k in range(1, ROW1 // 16):
                colk = iota // 8 + 2 * (k - 1)
                wk = w.at[colk].get(mode="promise_in_bounds")
                orow[i, pl.ds(16 * k, 16)] = wk * sr[i, pl.ds(16 * k, 16)]

    start_gather(0, 0)
    start_gather(1, 1)
    wait_gather(0)
    compute(0, 0)
    start_scatter(0, 0)
    start_gather(2, 0)
    wait_gather(1)
    compute(1, 1)
    start_scatter(1, 1)
    start_gather(3, 1)

    def steady(ihalf, c):
        ci = ihalf * 2
        for b in range(2):
            cib = ci + b
            wait_gather(b)
            wait_scatter(b)
            compute(cib, b)
            start_scatter(cib, b)
            start_gather(cib + 2, b)
        return c
    lax.fori_loop(1, (NCHUNK - 3) // 2, steady, 0)

    for cib in (NCHUNK - 3, NCHUNK - 2, NCHUNK - 1):
        b = cib % 2
        wait_gather(b)
        wait_scatter(b)
        compute(cib, b)
        start_scatter(cib, b)
        if cib + 2 < NCHUNK:
            start_gather(cib + 2, b)
    wait_scatter(0)
    wait_scatter(1)

    plsc.subcore_barrier()
    pltpu.sync_copy(accum.at[pl.ds(tid * RPT, RPT)],
                    out_hbm.at[cid, pl.ds(tid * RPT, RPT)])


def _make_sc_edge1(interpret=False):
    return functools.partial(
        pl.kernel,
        out_type=jax.ShapeDtypeStruct((NC, NPAD, ROW1), _f32),
        mesh=plsc.VectorSubcoreMesh(core_axis_name="c", subcore_axis_name="s",
                                    num_cores=NC, num_subcores=NS),
        scratch_types=[
            pltpu.VMEM((NCHUNK, CH), _i32),
            pltpu.VMEM((NCHUNK, CH), _i32),
            pltpu.VMEM((CH, ROW1), _f32),
            pltpu.VMEM((CH, ROW1), _f32),
            pltpu.VMEM((CH, ROW1), _f32),
            pltpu.VMEM((CH, ROW1), _f32),
            pltpu.VMEM((CH, 16), _f32),
            pltpu.VMEM((CH, 16), _f32),
            pltpu.VMEM((CH, ROW1), _f32),
            pltpu.VMEM_SHARED((NPAD, ROW1), _f32),
            pltpu.SemaphoreType.DMA,
            pltpu.SemaphoreType.DMA,
            pltpu.SemaphoreType.DMA,
            pltpu.SemaphoreType.DMA,
            pltpu.SemaphoreType.DMA,
            pltpu.SemaphoreType.DMA,
        ],
        compiler_params=_SC_PARAMS,
        interpret=interpret,
    )(_sc_edge1_body)


_sc_edge1 = _make_sc_edge1()


def _mid_body(part_ref, nodetab_ref, adsttab_ref, b1_ref, w2_ref,
              as2_ref, ad2_ref, nodetab2_ref):
    acc = part_ref[0, :N_NODES] + part_ref[1, :N_NODES]
    nodetab = nodetab_ref[...]
    asrc = nodetab[:, 0:8]
    h1 = nodetab[:, 16:ROW1]
    adst = adsttab_ref[...][:, 0:8]
    t = asrc + adst
    wself = jnp.exp(jnp.where(t > 0, t, 0.2 * t))
    den = acc[:, 0:8] + wself
    row = lax.broadcasted_iota(_i32, (8, 64), 0)
    col = lax.broadcasted_iota(_i32, (8, 64), 1)
    s8 = (row == col // 8).astype(_f32)
    num = acc[:, 16:ROW1] + jnp.dot(wself, s8, preferred_element_type=_f32) * h1
    dene = jnp.dot(den, s8, preferred_element_type=_f32)
    o = num / (dene + 1e-16) + b1_ref[...]
    x2 = jnp.where(o > 0, o, jnp.exp(o) - 1.0)
    h2 = jnp.dot(x2, w2_ref[...], preferred_element_type=_f32)
    as2 = jnp.dot(h2, as2_ref[...], preferred_element_type=_f32)
    ad2 = jnp.dot(h2, ad2_ref[...], preferred_element_type=_f32)
    z11 = jnp.zeros((h2.shape[0], 11), _f32)
    nodetab2_ref[...] = jnp.concatenate([as2, h2, ad2, z11], axis=1)


_mid = pl.pallas_call(
    _mid_body,
    out_shape=jax.ShapeDtypeStruct((N_NODES, ROW2), _f32),
)


def _sc_edge2_body(nodetab2, srcidx, dstidx, out_hbm,
                   sidx, didx, srows0, srows1, orows0, orows1, drows0, drows1,
                   zbuf, accum,
                   sga0, sga1, sgb0, sgb1, ssc0, ssc1):
    cid = lax.axis_index("c")
    tid = lax.axis_index("s")
    crow = (cid * NS + tid) * NCHUNK
    iota = lax.broadcasted_iota(_i32, (16,), 0)
    zidx = jnp.zeros((16,), _i32)
    srows = (srows0, srows1)
    orows = (orows0, orows1)
    drows = (drows0, drows1)
    sga = (sga0, sga1)
    sgb = (sgb0, sgb1)
    ssc = (ssc0, ssc1)

    _zero_stripe(zbuf, accum, tid, ROW2 // 16)
    plsc.subcore_barrier()

    pltpu.sync_copy(srcidx.at[pl.ds(crow, NCHUNK)], sidx)
    pltpu.sync_copy(dstidx.at[pl.ds(crow, NCHUNK)], didx)

    def start_gather(ci, b):
        pltpu.async_copy(nodetab2.at[sidx.at[ci]], srows[b], sga[b])
        pltpu.async_copy(nodetab2.at[didx.at[ci]], drows[b], sgb[b])

    def wait_gather(b):
        pltpu.make_async_copy(nodetab2.at[sidx.at[0]], srows[b], sga[b]).wait()
        pltpu.make_async_copy(nodetab2.at[didx.at[0]], drows[b], sgb[b]).wait()

    def start_scatter(ci, b):
        pltpu.async_copy(orows[b], accum.at[didx.at[ci]], ssc[b], add=True)

    def wait_scatter(b):
        pltpu.make_async_copy(orows[b], accum.at[didx.at[0]], ssc[b]).wait()

    def compute(ci, b):
        sr, orow, dr = srows[b], orows[b], drows[b]

        @plsc.parallel_loop(0, CH, unroll=8)
        def edge(i):
            s = sr[i, pl.ds(0, 16)]
            d = dr[i, pl.ds(0, 16)]
            advec = d.at[jnp.full((16,), 4, _i32)].get(mode="promise_in_bounds")
            t = s + advec
            t = jnp.where(t > 0, t, 0.2 * t)
            w = jnp.exp(t)
            wb = w.at[zidx].get(mode="promise_in_bounds")
            sm = jnp.where(iota == 0, 1.0, jnp.where(iota < 4, s, 0.0))
            orow[i, pl.ds(0, 16)] = wb * sm

    start_gather(0, 0)
    start_gather(1, 1)
    wait_gather(0)
    compute(0, 0)
    start_scatter(0, 0)
    start_gather(2, 0)
    wait_gather(1)
    compute(1, 1)
    start_scatter(1, 1)
    start_gather(3, 1)

    def steady(ihalf, c):
        ci = ihalf * 2
        for b in range(2):
            cib = ci + b
            wait_gather(b)
            wait_scatter(b)
            compute(cib, b)
            start_scatter(cib, b)
            start_gather(cib + 2, b)
        return c
    lax.fori_loop(1, (NCHUNK - 3) // 2, steady, 0)

    for cib in (NCHUNK - 3, NCHUNK - 2, NCHUNK - 1):
        b = cib % 2
        wait_gather(b)
        wait_scatter(b)
        compute(cib, b)
        start_scatter(cib, b)
        if cib + 2 < NCHUNK:
            start_gather(cib + 2, b)
    wait_scatter(0)
    wait_scatter(1)

    plsc.subcore_barrier()
    pltpu.sync_copy(accum.at[pl.ds(tid * RPT, RPT)],
                    out_hbm.at[cid, pl.ds(tid * RPT, RPT)])


def _make_sc_edge2(interpret=False):
    return functools.partial(
        pl.kernel,
        out_type=jax.ShapeDtypeStruct((NC, NPAD, ROW2), _f32),
        mesh=plsc.VectorSubcoreMesh(core_axis_name="c", subcore_axis_name="s",
                                    num_cores=NC, num_subcores=NS),
        scratch_types=[
            pltpu.VMEM((NCHUNK, CH), _i32),
            pltpu.VMEM((NCHUNK, CH), _i32),
            pltpu.VMEM((CH, ROW2), _f32),
            pltpu.VMEM((CH, ROW2), _f32),
            pltpu.VMEM((CH, ROW2), _f32),
            pltpu.VMEM((CH, ROW2), _f32),
            pltpu.VMEM((CH, ROW2), _f32),
            pltpu.VMEM((CH, ROW2), _f32),
            pltpu.VMEM((CH, ROW2), _f32),
            pltpu.VMEM_SHARED((NPAD, ROW2), _f32),
            pltpu.SemaphoreType.DMA,
            pltpu.SemaphoreType.DMA,
            pltpu.SemaphoreType.DMA,
            pltpu.SemaphoreType.DMA,
            pltpu.SemaphoreType.DMA,
            pltpu.SemaphoreType.DMA,
        ],
        compiler_params=_SC_PARAMS,
        interpret=interpret,
    )(_sc_edge2_body)


_sc_edge2 = _make_sc_edge2()


def _fin_body(part2_ref, nodetab2_ref, b2_ref, out_ref):
    acc = part2_ref[0, :N_NODES] + part2_ref[1, :N_NODES]
    tab = nodetab2_ref[...]
    as2 = tab[:, 0:1]
    h2 = tab[:, 1:4]
    ad2 = tab[:, 4:5]
    t = as2 + ad2
    w = jnp.exp(jnp.where(t > 0, t, 0.2 * t))
    den = acc[:, 0:1] + w
    num = acc[:, 1:4] + w * h2
    out_ref[...] = num / (den + 1e-16) + b2_ref[...]


_fin = pl.pallas_call(
    _fin_body,
    out_shape=jax.ShapeDtypeStruct((N_NODES, 3), _f32),
)


@jax.jit
def kernel(x, edge_index, W1, att_src1, att_dst1, b1, W2, att_src2, att_dst2, b2):
    ei = edge_index.astype(_i32)
    src2d = ei[0].reshape(NROWS, CH)
    dst2d = ei[1].reshape(NROWS, CH)
    nodetab, adsttab = _prep1(x, W1, att_src1.reshape(1, 64),
                              att_dst1.reshape(1, 64))
    part1 = _sc_edge1(nodetab, adsttab, src2d, dst2d)
    nodetab2 = _mid(part1, nodetab, adsttab, b1.reshape(1, 64), W2,
                    att_src2.reshape(3, 1), att_dst2.reshape(3, 1))
    part2 = _sc_edge2(nodetab2, src2d, dst2d)
    return _fin(part2, nodetab2, b2.reshape(1, 3))


# speedup vs baseline: 217.8451x; 1.1087x over previous
import functools

import jax
import jax.numpy as jnp
from jax import lax
from jax.experimental import pallas as pl
from jax.experimental.pallas import tpu as pltpu
from jax.experimental.pallas import tpu_sc as plsc

N_NODES = 10000
NPAD = 10240
N_EDGES = 320000
NC, NS = 2, 16
NW = NC * NS
EPT = N_EDGES // NW
CH = 80
NCHUNK = EPT // CH
NROWS = N_EDGES // CH
RPT = NPAD // NS
ROW1 = 80
ROW2 = 16

_f32 = jnp.float32
_i32 = jnp.int32

_SC_PARAMS = pltpu.CompilerParams(use_tc_tiling_on_sc=False,
                                  needs_layout_passes=False)


def _prep1_body(x_ref, w1_ref, asrc_ref, adst_ref, nodetab_ref, adsttab_ref):
    h = jnp.dot(x_ref[...], w1_ref[...], preferred_element_type=_f32)
    row = lax.broadcasted_iota(_i32, (64, 8), 0)
    col = lax.broadcasted_iota(_i32, (64, 8), 1)
    g = (row // 8 == col).astype(_f32)
    asrc = jnp.dot(h * asrc_ref[...], g, preferred_element_type=_f32)
    adst = jnp.dot(h * adst_ref[...], g, preferred_element_type=_f32)
    z8 = jnp.zeros_like(asrc)
    nodetab_ref[...] = jnp.concatenate([asrc, z8, h], axis=1)
    adsttab_ref[...] = jnp.concatenate([adst, z8], axis=1)


_prep1 = pl.pallas_call(
    _prep1_body,
    out_shape=(
        jax.ShapeDtypeStruct((N_NODES, ROW1), _f32),
        jax.ShapeDtypeStruct((N_NODES, 16), _f32),
    ),
)


def _zero_stripe(zbuf, accum, tid, rowbytes16):
    zeros16 = jnp.zeros((16,), _f32)

    def zrow(r, c):
        for k in range(rowbytes16):
            zbuf[r, pl.ds(k * 16, 16)] = zeros16
        return c
    lax.fori_loop(0, CH, zrow, 0)
    for j in range(RPT // CH):
        pltpu.sync_copy(zbuf, accum.at[pl.ds(tid * RPT + j * CH, CH)])


def _sc_edge1_body(nodetab, adsttab, srcidx, dstidx, out_hbm,
                   sidx, didx, srows0, srows1, orows0, orows1, drows0, drows1,
                   zbuf, accum,
                   sga0, sga1, sgb0, sgb1, ssc0, ssc1):
    cid = lax.axis_index("c")
    tid = lax.axis_index("s")
    crow = (cid * NS + tid) * NCHUNK
    iota = lax.broadcasted_iota(_i32, (16,), 0)
    srows = (srows0, srows1)
    orows = (orows0, orows1)
    drows = (drows0, drows1)
    sga = (sga0, sga1)
    sgb = (sgb0, sgb1)
    ssc = (ssc0, ssc1)

    _zero_stripe(zbuf, accum, tid, ROW1 // 16)
    plsc.subcore_barrier()

    pltpu.sync_copy(srcidx.at[pl.ds(crow, NCHUNK)], sidx)
    pltpu.sync_copy(dstidx.at[pl.ds(crow, NCHUNK)], didx)

    def start_gather(ci, b):
        pltpu.async_copy(nodetab.at[sidx.at[ci]], srows[b], sga[b])
        pltpu.async_copy(adsttab.at[didx.at[ci]], drows[b], sgb[b])

    def wait_gather(b):
        pltpu.make_async_copy(nodetab.at[sidx.at[0]], srows[b], sga[b]).wait()
        pltpu.make_async_copy(adsttab.at[didx.at[0]], drows[b], sgb[b]).wait()

    def start_scatter(ci, b):
        pltpu.async_copy(orows[b], accum.at[didx.at[ci]], ssc[b], add=True)

    def wait_scatter(b):
        pltpu.make_async_copy(orows[b], accum.at[didx.at[0]], ssc[b]).wait()

    def compute(ci, b):
        sr, orow, dr = srows[b], orows[b], drows[b]

        @plsc.parallel_loop(0, CH, unroll=8)
        def edge(i):
            r0 = sr[i, pl.ds(0, 16)]
            d0 = dr[i, pl.ds(0, 16)]
            t = r0 + d0
            t = jnp.where(t > 0, t, 0.2 * t)
            w = jnp.exp(t)
            orow[i, pl.ds(0, 16)] = w
            for k in range(1, ROW1 // 16):
                colk = iota // 8 + 2 * (k - 1)
                wk = w.at[colk].get(mode="promise_in_bounds")
                orow[i, pl.ds(16 * k, 16)] = wk * sr[i, pl.ds(16 * k, 16)]

    start_gather(0, 0)
    start_gather(1, 1)
    wait_gather(0)
    compute(0, 0)
    start_scatter(0, 0)
    start_gather(2, 0)
    wait_gather(1)
    compute(1, 1)
    start_scatter(1, 1)
    start_gather(3, 1)

    def steady(ihalf, c):
        ci = ihalf * 2
        for b in range(2):
            cib = ci + b
            wait_gather(b)
            wait_scatter(b)
            compute(cib, b)
            start_scatter(cib, b)
            start_gather(cib + 2, b)
        return c
    lax.fori_loop(1, (NCHUNK - 3) // 2, steady, 0)

    for cib in (NCHUNK - 3, NCHUNK - 2, NCHUNK - 1):
        b = cib % 2
        wait_gather(b)
        wait_scatter(b)
        compute(cib, b)
        start_scatter(cib, b)
        if cib + 2 < NCHUNK:
            start_gather(cib + 2, b)
    wait_scatter(0)
    wait_scatter(1)

    plsc.subcore_barrier()
    pltpu.sync_copy(accum.at[pl.ds(tid * RPT, RPT)],
                    out_hbm.at[cid, pl.ds(tid * RPT, RPT)])


def _make_sc_edge1(interpret=False):
    return functools.partial(
        pl.kernel,
        out_type=jax.ShapeDtypeStruct((NC, NPAD, ROW1), _f32),
        mesh=plsc.VectorSubcoreMesh(core_axis_name="c", subcore_axis_name="s",
                                    num_cores=NC, num_subcores=NS),
        scratch_types=[
            pltpu.VMEM((NCHUNK, CH), _i32),
            pltpu.VMEM((NCHUNK, CH), _i32),
            pltpu.VMEM((CH, ROW1), _f32),
            pltpu.VMEM((CH, ROW1), _f32),
            pltpu.VMEM((CH, ROW1), _f32),
            pltpu.VMEM((CH, ROW1), _f32),
            pltpu.VMEM((CH, 16), _f32),
            pltpu.VMEM((CH, 16), _f32),
            pltpu.VMEM((CH, ROW1), _f32),
            pltpu.VMEM_SHARED((NPAD, ROW1), _f32),
            pltpu.SemaphoreType.DMA,
            pltpu.SemaphoreType.DMA,
            pltpu.SemaphoreType.DMA,
            pltpu.SemaphoreType.DMA,
            pltpu.SemaphoreType.DMA,
            pltpu.SemaphoreType.DMA,
        ],
        compiler_params=_SC_PARAMS,
        interpret=interpret,
    )(_sc_edge1_body)


_sc_edge1 = _make_sc_edge1()


def _mid_body(part_ref, nodetab_ref, adsttab_ref, b1_ref, w2_ref,
              as2_ref, ad2_ref, nodetab2_ref):
    acc = part_ref[0, :N_NODES] + part_ref[1, :N_NODES]
    nodetab = nodetab_ref[...]
    asrc = nodetab[:, 0:8]
    h1 = nodetab[:, 16:ROW1]
    adst = adsttab_ref[...][:, 0:8]
    t = asrc + adst
    wself = jnp.exp(jnp.where(t > 0, t, 0.2 * t))
    den = acc[:, 0:8] + wself
    row = lax.broadcasted_iota(_i32, (8, 64), 0)
    col = lax.broadcasted_iota(_i32, (8, 64), 1)
    s8 = (row == col // 8).astype(_f32)
    num = acc[:, 16:ROW1] + jnp.dot(wself, s8, preferred_element_type=_f32) * h1
    dene = jnp.dot(den, s8, preferred_element_type=_f32)
    o = num / (dene + 1e-16) + b1_ref[...]
    x2 = jnp.where(o > 0, o, jnp.exp(o) - 1.0)
    h2 = jnp.dot(x2, w2_ref[...], preferred_element_type=_f32)
    as2 = jnp.dot(h2, as2_ref[...], preferred_element_type=_f32)
    ad2 = jnp.dot(h2, ad2_ref[...], preferred_element_type=_f32)
    z11 = jnp.zeros((h2.shape[0], 11), _f32)
    nodetab2_ref[...] = jnp.concatenate([as2, h2, ad2, z11], axis=1)


_mid = pl.pallas_call(
    _mid_body,
    out_shape=jax.ShapeDtypeStruct((N_NODES, ROW2), _f32),
)


K2 = 5
NCH2 = NCHUNK // K2


def _sc_edge2_body(nodetab2, srcidx, dstidx, out_hbm,
                   sidx, didx, srows0, srows1, orows0, orows1, drows0, drows1,
                   zbuf, accum,
                   sga0, sga1, sgb0, sgb1, ssc0, ssc1):
    cid = lax.axis_index("c")
    tid = lax.axis_index("s")
    crow = (cid * NS + tid) * NCHUNK
    iota = lax.broadcasted_iota(_i32, (16,), 0)
    zidx = jnp.zeros((16,), _i32)
    srows = (srows0, srows1)
    orows = (orows0, orows1)
    drows = (drows0, drows1)
    sga = (sga0, sga1)
    sgb = (sgb0, sgb1)
    ssc = (ssc0, ssc1)

    _zero_stripe(zbuf, accum, tid, ROW2 // 16)
    plsc.subcore_barrier()

    pltpu.sync_copy(srcidx.at[pl.ds(crow, NCHUNK)], sidx)
    pltpu.sync_copy(dstidx.at[pl.ds(crow, NCHUNK)], didx)

    def start_gather(ci, b):
        for j in range(K2):
            pltpu.async_copy(nodetab2.at[sidx.at[ci * K2 + j]],
                             srows[b].at[pl.ds(j * CH, CH)], sga[b])
            pltpu.async_copy(nodetab2.at[didx.at[ci * K2 + j]],
                             drows[b].at[pl.ds(j * CH, CH)], sgb[b])

    def wait_gather(b):
        for j in range(K2):
            pltpu.make_async_copy(nodetab2.at[sidx.at[0]],
                                  srows[b].at[pl.ds(0, CH)], sga[b]).wait()
            pltpu.make_async_copy(nodetab2.at[didx.at[0]],
                                  drows[b].at[pl.ds(0, CH)], sgb[b]).wait()

    def start_scatter(ci, b):
        for j in range(K2):
            pltpu.async_copy(orows[b].at[pl.ds(j * CH, CH)],
                             accum.at[didx.at[ci * K2 + j]], ssc[b], add=True)

    def wait_scatter(b):
        for j in range(K2):
            pltpu.make_async_copy(orows[b].at[pl.ds(0, CH)],
                                  accum.at[didx.at[0]], ssc[b]).wait()

    def compute(ci, b):
        sr, orow, dr = srows[b], orows[b], drows[b]

        @plsc.parallel_loop(0, K2 * CH, unroll=8)
        def edge(i):
            s = sr[i, pl.ds(0, 16)]
            d = dr[i, pl.ds(0, 16)]
            advec = d.at[jnp.full((16,), 4, _i32)].get(mode="promise_in_bounds")
            t = s + advec
            t = jnp.where(t > 0, t, 0.2 * t)
            w = jnp.exp(t)
            wb = w.at[zidx].get(mode="promise_in_bounds")
            sm = jnp.where(iota == 0, 1.0, jnp.where(iota < 4, s, 0.0))
            orow[i, pl.ds(0, 16)] = wb * sm

    start_gather(0, 0)
    start_gather(1, 1)
    wait_gather(0)
    compute(0, 0)
    start_scatter(0, 0)
    start_gather(2, 0)
    wait_gather(1)
    compute(1, 1)
    start_scatter(1, 1)
    start_gather(3, 1)

    def steady(ihalf, c):
        ci = ihalf * 2
        for b in range(2):
            cib = ci + b
            wait_gather(b)
            wait_scatter(b)
            compute(cib, b)
            start_scatter(cib, b)
            start_gather(cib + 2, b)
        return c
    lax.fori_loop(1, (NCH2 - 3) // 2, steady, 0)

    for cib in (NCH2 - 3, NCH2 - 2, NCH2 - 1):
        b = cib % 2
        wait_gather(b)
        wait_scatter(b)
        compute(cib, b)
        start_scatter(cib, b)
        if cib + 2 < NCH2:
            start_gather(cib + 2, b)
    wait_scatter(0)
    wait_scatter(1)

    plsc.subcore_barrier()
    pltpu.sync_copy(accum.at[pl.ds(tid * RPT, RPT)],
                    out_hbm.at[cid, pl.ds(tid * RPT, RPT)])


def _make_sc_edge2(interpret=False):
    return functools.partial(
        pl.kernel,
        out_type=jax.ShapeDtypeStruct((NC, NPAD, ROW2), _f32),
        mesh=plsc.VectorSubcoreMesh(core_axis_name="c", subcore_axis_name="s",
                                    num_cores=NC, num_subcores=NS),
        scratch_types=[
            pltpu.VMEM((NCHUNK, CH), _i32),
            pltpu.VMEM((NCHUNK, CH), _i32),
            pltpu.VMEM((K2 * CH, ROW2), _f32),
            pltpu.VMEM((K2 * CH, ROW2), _f32),
            pltpu.VMEM((K2 * CH, ROW2), _f32),
            pltpu.VMEM((K2 * CH, ROW2), _f32),
            pltpu.VMEM((K2 * CH, ROW2), _f32),
            pltpu.VMEM((K2 * CH, ROW2), _f32),
            pltpu.VMEM((CH, ROW2), _f32),
            pltpu.VMEM_SHARED((NPAD, ROW2), _f32),
            pltpu.SemaphoreType.DMA,
            pltpu.SemaphoreType.DMA,
            pltpu.SemaphoreType.DMA,
            pltpu.SemaphoreType.DMA,
            pltpu.SemaphoreType.DMA,
            pltpu.SemaphoreType.DMA,
        ],
        compiler_params=_SC_PARAMS,
        interpret=interpret,
    )(_sc_edge2_body)


_sc_edge2 = _make_sc_edge2()


def _fin_body(part2_ref, nodetab2_ref, b2_ref, out_ref):
    acc = part2_ref[0, :N_NODES] + part2_ref[1, :N_NODES]
    tab = nodetab2_ref[...]
    as2 = tab[:, 0:1]
    h2 = tab[:, 1:4]
    ad2 = tab[:, 4:5]
    t = as2 + ad2
    w = jnp.exp(jnp.where(t > 0, t, 0.2 * t))
    den = acc[:, 0:1] + w
    num = acc[:, 1:4] + w * h2
    out_ref[...] = num / (den + 1e-16) + b2_ref[...]


_fin = pl.pallas_call(
    _fin_body,
    out_shape=jax.ShapeDtypeStruct((N_NODES, 3), _f32),
)


@jax.jit
def kernel(x, edge_index, W1, att_src1, att_dst1, b1, W2, att_src2, att_dst2, b2):
    ei = edge_index.astype(_i32)
    src2d = ei[0].reshape(NROWS, CH)
    dst2d = ei[1].reshape(NROWS, CH)
    nodetab, adsttab = _prep1(x, W1, att_src1.reshape(1, 64),
                              att_dst1.reshape(1, 64))
    part1 = _sc_edge1(nodetab, adsttab, src2d, dst2d)
    nodetab2 = _mid(part1, nodetab, adsttab, b1.reshape(1, 64), W2,
                    att_src2.reshape(3, 1), att_dst2.reshape(3, 1))
    part2 = _sc_edge2(nodetab2, src2d, dst2d)
    return _fin(part2, nodetab2, b2.reshape(1, 3))
